# baseline (speedup 1.0000x reference)
import jax
import jax.numpy as jnp
import numpy as np
from jax.experimental import pallas as pl
from jax.experimental.pallas import tpu as pltpu
from jax.experimental.pallas import tpu_sc as plsc

_shard_map = getattr(jax, "shard_map", None)
if _shard_map is None:
    from jax.experimental.shard_map import shard_map as _shard_map

_TM = 4096
_TN = 640
_TP = 3200
_GW = 128
_GC = 128


def _gather_rows(table, idx2d, n_idx):
    mesh = plsc.VectorSubcoreMesh(core_axis_name="c", subcore_axis_name="s")

    @pl.kernel(
        out_type=jax.ShapeDtypeStruct((n_idx, _GC), table.dtype),
        mesh=mesh,
    )
    def gather_kernel(x_hbm, i_hbm, o_hbm):
        def body(i_vmem, o_vmem):
            pltpu.sync_copy(x_hbm.at[i_vmem.at[0]], o_vmem)

        pltpu.emit_pipeline(
            body,
            grid=(n_idx // _GW,),
            in_specs=[pl.BlockSpec((1, _GW), lambda i: (0, i))],
            out_specs=[pl.BlockSpec((_GW, _GC), lambda i: (i, 0))],
            core_axis_name=("c", "s"),
            dimension_semantics=(pltpu.PARALLEL,),
        )(i_hbm, o_hbm)

    return gather_kernel(table, idx2d)


def _precompute_kernel(w2_ref, w1_ref, b1_ref, b2_ref, m_ref, b2e_ref):
    w2 = w2_ref[...]
    m = jax.lax.dot_general(
        w2, w1_ref[...], (((1,), (0,)), ((), ())),
        preferred_element_type=jnp.float32,
    )
    m_ref[...] = m.astype(jnp.bfloat16)
    b2e_ref[...] = b2_ref[...] + jax.lax.dot_general(
        b1_ref[...], w2, (((1,), (1,)), ((), ())),
        preferred_element_type=jnp.float32,
    )


def _precompute(w2, w1, b1_2d, b2_2d, n_rows, d_model):
    return pl.pallas_call(
        _precompute_kernel,
        grid=(n_rows // _TP,),
        in_specs=[
            pl.BlockSpec((_TP, d_model), lambda p: (p, 0)),
            pl.BlockSpec((d_model, d_model), lambda p: (0, 0)),
            pl.BlockSpec((1, d_model), lambda p: (0, 0)),
            pl.BlockSpec((1, _TP), lambda p: (0, p)),
        ],
        out_specs=[
            pl.BlockSpec((_TP, d_model), lambda p: (p, 0)),
            pl.BlockSpec((1, _TP), lambda p: (0, p)),
        ],
        out_shape=[
            jax.ShapeDtypeStruct((n_rows, d_model), jnp.bfloat16),
            jax.ShapeDtypeStruct((1, n_rows), jnp.float32),
        ],
        compiler_params=pltpu.CompilerParams(
            dimension_semantics=("arbitrary",),
        ),
    )(w2, w1, b1_2d, b2_2d)


def _head_kernel(h0_ref, m_ref, b2e_ref, out_ref):
    out_ref[...] = jax.lax.dot_general(
        h0_ref[...], m_ref[...], (((1,), (1,)), ((), ())),
        preferred_element_type=jnp.float32,
    ) + b2e_ref[...]


def _head(h0, m, b2e, n_tok, d_model, n_cols):
    return pl.pallas_call(
        _head_kernel,
        grid=(n_tok // _TM, n_cols // _TN),
        in_specs=[
            pl.BlockSpec((_TM, d_model), lambda t, v: (t, 0)),
            pl.BlockSpec((_TN, d_model), lambda t, v: (v, 0)),
            pl.BlockSpec((1, _TN), lambda t, v: (0, v)),
        ],
        out_specs=pl.BlockSpec((_TM, _TN), lambda t, v: (t, v)),
        out_shape=jax.ShapeDtypeStruct((n_tok, n_cols), jnp.float32),
        compiler_params=pltpu.CompilerParams(
            dimension_semantics=("arbitrary", "arbitrary"),
        ),
    )(h0, m, b2e)


def _forward_1dev(x_flat, embed, W1, b1, W2, b2):
    (n_tok,) = x_flat.shape
    vocab, d_model = embed.shape
    rows_per_tok = d_model // _GC
    idx = (x_flat.reshape(n_tok, 1).astype(jnp.int32) * rows_per_tok
           + jnp.arange(rows_per_tok, dtype=jnp.int32))
    h0 = _gather_rows(
        embed.reshape(vocab * rows_per_tok, _GC),
        idx.reshape(1, n_tok * rows_per_tok),
        n_tok * rows_per_tok,
    ).reshape(n_tok, d_model)
    m, b2e = _precompute(W2, W1, b1.reshape(1, d_model), b2.reshape(1, vocab),
                         vocab, d_model)
    return _head(h0.astype(jnp.bfloat16), m, b2e, n_tok, d_model, vocab)


def _forward_2dev(x_flat, embed_half, W1, b1, W2_half, b2_half):
    (n_tok,) = x_flat.shape
    half_vocab, d_model = embed_half.shape
    voff = jax.lax.axis_index("d").astype(jnp.int32) * half_vocab
    xi = x_flat.astype(jnp.int32)
    local = xi - voff
    valid = (local >= 0) & (local < half_vocab)
    safe = jnp.clip(local, 0, half_vocab - 1)
    rows_per_tok = d_model // _GC
    idx = (safe.reshape(n_tok, 1) * rows_per_tok
           + jnp.arange(rows_per_tok, dtype=jnp.int32))
    h0p = _gather_rows(
        embed_half.reshape(half_vocab * rows_per_tok, _GC),
        idx.reshape(1, n_tok * rows_per_tok),
        n_tok * rows_per_tok,
    ).reshape(n_tok, d_model)
    h0p = jnp.where(valid[:, None], h0p, 0.0).astype(jnp.bfloat16)
    h0 = jax.lax.psum(h0p, "d")
    m, b2e = _precompute(W2_half, W1, b1.reshape(1, d_model),
                         b2_half.reshape(1, half_vocab),
                         half_vocab, d_model)
    return _head(h0, m, b2e, n_tok, d_model, half_vocab)


def kernel(x, embed, W1, b1, W2, b2):
    b, s = x.shape
    vocab, d_model = embed.shape
    n_tok = b * s
    devs = jax.devices()
    if len(devs) < 2 or vocab % 2 or (vocab // 2) % (2 * _TP):
        logits = _forward_1dev(x.reshape(n_tok), embed, W1, b1, W2, b2)
        return logits.reshape(b, s, vocab)
    mesh = jax.sharding.Mesh(np.array(devs[:2]), ("d",))
    p = jax.sharding.PartitionSpec
    logits = _shard_map(
        _forward_2dev,
        mesh=mesh,
        in_specs=(p(None), p("d", None), p(None, None), p(None),
                  p("d", None), p("d")),
        out_specs=p(None, "d"),
        check_vma=False,
    )(x.reshape(n_tok), embed, W1, b1, W2, b2)
    return logits.reshape(b, s, vocab)


# speedup vs baseline: 1.0672x; 1.0672x over previous
import jax
import jax.numpy as jnp
import numpy as np
from jax.experimental import pallas as pl
from jax.experimental.pallas import tpu as pltpu
from jax.experimental.pallas import tpu_sc as plsc

_shard_map = getattr(jax, "shard_map", None)
if _shard_map is None:
    from jax.experimental.shard_map import shard_map as _shard_map

_TM = 2048
_TN = 1280
_TM2 = 1024
_TN2 = 3200
_TP = 3200
_GW = 128
_GC = 128


def _gather_rows(table, idx2d, n_idx):
    mesh = plsc.VectorSubcoreMesh(core_axis_name="c", subcore_axis_name="s")

    @pl.kernel(
        out_type=jax.ShapeDtypeStruct((n_idx, _GC), table.dtype),
        mesh=mesh,
    )
    def gather_kernel(x_hbm, i_hbm, o_hbm):
        def body(i_vmem, o_vmem):
            pltpu.sync_copy(x_hbm.at[i_vmem.at[0]], o_vmem)

        pltpu.emit_pipeline(
            body,
            grid=(n_idx // _GW,),
            in_specs=[pl.BlockSpec((1, _GW), lambda i: (0, i))],
            out_specs=[pl.BlockSpec((_GW, _GC), lambda i: (i, 0))],
            core_axis_name=("c", "s"),
            dimension_semantics=(pltpu.PARALLEL,),
        )(i_hbm, o_hbm)

    return gather_kernel(table, idx2d)


def _precompute_kernel(w2_ref, w1_ref, b1_ref, b2_ref, m_ref, b2e_ref):
    w2 = w2_ref[...]
    m = jax.lax.dot_general(
        w2, w1_ref[...], (((1,), (0,)), ((), ())),
        preferred_element_type=jnp.float32,
    )
    m_ref[...] = m.astype(jnp.bfloat16)
    b2e_ref[...] = b2_ref[...] + jax.lax.dot_general(
        b1_ref[...], w2, (((1,), (1,)), ((), ())),
        preferred_element_type=jnp.float32,
    )


def _precompute(w2, w1, b1_2d, b2_2d, n_rows, d_model):
    return pl.pallas_call(
        _precompute_kernel,
        grid=(n_rows // _TP,),
        in_specs=[
            pl.BlockSpec((_TP, d_model), lambda p: (p, 0)),
            pl.BlockSpec((d_model, d_model), lambda p: (0, 0)),
            pl.BlockSpec((1, d_model), lambda p: (0, 0)),
            pl.BlockSpec((1, _TP), lambda p: (0, p)),
        ],
        out_specs=[
            pl.BlockSpec((_TP, d_model), lambda p: (p, 0)),
            pl.BlockSpec((1, _TP), lambda p: (0, p)),
        ],
        out_shape=[
            jax.ShapeDtypeStruct((n_rows, d_model), jnp.bfloat16),
            jax.ShapeDtypeStruct((1, n_rows), jnp.float32),
        ],
        compiler_params=pltpu.CompilerParams(
            dimension_semantics=("arbitrary",),
        ),
    )(w2, w1, b1_2d, b2_2d)


def _head_kernel(h0_ref, m_ref, b2e_ref, out_ref):
    out_ref[...] = jax.lax.dot_general(
        h0_ref[...], m_ref[...], (((1,), (1,)), ((), ())),
        preferred_element_type=jnp.float32,
    ) + b2e_ref[...]


def _head(h0, m, b2e, n_tok, d_model, n_cols):
    return pl.pallas_call(
        _head_kernel,
        grid=(n_tok // _TM, n_cols // _TN),
        in_specs=[
            pl.BlockSpec((_TM, d_model), lambda t, v: (t, 0)),
            pl.BlockSpec((_TN, d_model), lambda t, v: (v, 0)),
            pl.BlockSpec((1, _TN), lambda t, v: (0, v)),
        ],
        out_specs=pl.BlockSpec((_TM, _TN), lambda t, v: (t, v)),
        out_shape=jax.ShapeDtypeStruct((n_tok, n_cols), jnp.float32),
        compiler_params=pltpu.CompilerParams(
            dimension_semantics=("arbitrary", "arbitrary"),
        ),
    )(h0, m, b2e)


def _head2(h0, m, b2e, n_tok, d_model, n_cols):
    return pl.pallas_call(
        _head_kernel,
        grid=(n_cols // _TN2, n_tok // _TM2),
        in_specs=[
            pl.BlockSpec((_TM2, d_model), lambda v, t: (t, 0)),
            pl.BlockSpec((_TN2, d_model), lambda v, t: (v, 0)),
            pl.BlockSpec((1, _TN2), lambda v, t: (0, v)),
        ],
        out_specs=pl.BlockSpec((_TM2, _TN2), lambda v, t: (t, v)),
        out_shape=jax.ShapeDtypeStruct((n_tok, n_cols), jnp.float32),
        compiler_params=pltpu.CompilerParams(
            dimension_semantics=("arbitrary", "arbitrary"),
        ),
    )(h0, m, b2e)


def _forward_1dev(x_flat, embed, W1, b1, W2, b2):
    (n_tok,) = x_flat.shape
    vocab, d_model = embed.shape
    rows_per_tok = d_model // _GC
    idx = (x_flat.reshape(n_tok, 1).astype(jnp.int32) * rows_per_tok
           + jnp.arange(rows_per_tok, dtype=jnp.int32))
    h0 = _gather_rows(
        embed.reshape(vocab * rows_per_tok, _GC),
        idx.reshape(1, n_tok * rows_per_tok),
        n_tok * rows_per_tok,
    ).reshape(n_tok, d_model)
    m, b2e = _precompute(W2, W1, b1.reshape(1, d_model), b2.reshape(1, vocab),
                         vocab, d_model)
    return _head(h0.astype(jnp.bfloat16), m, b2e, n_tok, d_model, vocab)


def _forward_2dev(x_flat, embed_half, W1, b1, W2_half, b2_half):
    (n_tok,) = x_flat.shape
    half_vocab, d_model = embed_half.shape
    voff = jax.lax.axis_index("d").astype(jnp.int32) * half_vocab
    xi = x_flat.astype(jnp.int32)
    local = xi - voff
    valid = (local >= 0) & (local < half_vocab)
    safe = jnp.clip(local, 0, half_vocab - 1)
    rows_per_tok = d_model // _GC
    idx = (safe.reshape(n_tok, 1) * rows_per_tok
           + jnp.arange(rows_per_tok, dtype=jnp.int32))
    h0p = _gather_rows(
        embed_half.reshape(half_vocab * rows_per_tok, _GC),
        idx.reshape(1, n_tok * rows_per_tok),
        n_tok * rows_per_tok,
    ).reshape(n_tok, d_model)
    h0p = jnp.where(valid[:, None], h0p, 0.0).astype(jnp.bfloat16)
    h0 = jax.lax.psum(h0p, "d")
    m, b2e = _precompute(W2_half, W1, b1.reshape(1, d_model),
                         b2_half.reshape(1, half_vocab),
                         half_vocab, d_model)
    return _head2(h0, m, b2e, n_tok, d_model, half_vocab)


def kernel(x, embed, W1, b1, W2, b2):
    b, s = x.shape
    vocab, d_model = embed.shape
    n_tok = b * s
    devs = jax.devices()
    half = vocab // 2
    shardable = (len(devs) >= 2 and vocab % 2 == 0 and half % _TP == 0
                 and half % _TN2 == 0 and n_tok % _TM2 == 0)
    if not shardable:
        logits = _forward_1dev(x.reshape(n_tok), embed, W1, b1, W2, b2)
        return logits.reshape(b, s, vocab)
    mesh = jax.sharding.Mesh(np.array(devs[:2]), ("d",))
    p = jax.sharding.PartitionSpec
    logits = _shard_map(
        _forward_2dev,
        mesh=mesh,
        in_specs=(p(None), p("d", None), p(None, None), p(None),
                  p("d", None), p("d")),
        out_specs=p(None, "d"),
        check_vma=False,
    )(x.reshape(n_tok), embed, W1, b1, W2, b2)
    return logits.reshape(b, s, vocab)


# speedup vs baseline: 1.2765x; 1.1961x over previous
import jax
import jax.numpy as jnp
import numpy as np
from jax.experimental import pallas as pl
from jax.experimental.pallas import tpu as pltpu
from jax.experimental.pallas import tpu_sc as plsc

_shard_map = getattr(jax, "shard_map", None)
if _shard_map is None:
    from jax.experimental.shard_map import shard_map as _shard_map

_TM = 2048
_TN = 1280
_TM2 = 1024
_TN2 = 3200
_TP = 3200
_GW = 128
_GC = 128


def _gather_rows(table, idx2d, n_idx, width):
    mesh = plsc.VectorSubcoreMesh(core_axis_name="c", subcore_axis_name="s")
    gw = 32

    @pl.kernel(
        out_type=jax.ShapeDtypeStruct((n_idx, width), table.dtype),
        mesh=mesh,
    )
    def gather_kernel(x_hbm, i_hbm, o_hbm):
        def body(i_vmem, o_vmem):
            pltpu.sync_copy(x_hbm.at[i_vmem.at[0]], o_vmem)

        pltpu.emit_pipeline(
            body,
            grid=(n_idx // gw,),
            in_specs=[pl.BlockSpec((1, gw), lambda i: (i, 0))],
            out_specs=[pl.BlockSpec((gw, width), lambda i: (i, 0))],
            core_axis_name=("c", "s"),
            dimension_semantics=(pltpu.PARALLEL,),
        )(i_hbm, o_hbm)

    return gather_kernel(table, idx2d)


def _precompute_kernel(w2_ref, w1_ref, b1_ref, b2_ref, m_ref, b2e_ref):
    w2 = w2_ref[...]
    m = jax.lax.dot_general(
        w2, w1_ref[...], (((1,), (0,)), ((), ())),
        preferred_element_type=jnp.float32,
    )
    m_ref[...] = m.astype(jnp.bfloat16)
    b2e_ref[...] = b2_ref[...] + jax.lax.dot_general(
        b1_ref[...], w2, (((1,), (1,)), ((), ())),
        preferred_element_type=jnp.float32,
    )


def _precompute(w2, w1, b1_2d, b2_2d, n_rows, d_model):
    return pl.pallas_call(
        _precompute_kernel,
        grid=(n_rows // _TP,),
        in_specs=[
            pl.BlockSpec((_TP, d_model), lambda p: (p, 0)),
            pl.BlockSpec((d_model, d_model), lambda p: (0, 0)),
            pl.BlockSpec((1, d_model), lambda p: (0, 0)),
            pl.BlockSpec((1, _TP), lambda p: (0, p)),
        ],
        out_specs=[
            pl.BlockSpec((_TP, d_model), lambda p: (p, 0)),
            pl.BlockSpec((1, _TP), lambda p: (0, p)),
        ],
        out_shape=[
            jax.ShapeDtypeStruct((n_rows, d_model), jnp.bfloat16),
            jax.ShapeDtypeStruct((1, n_rows), jnp.float32),
        ],
        compiler_params=pltpu.CompilerParams(
            dimension_semantics=("arbitrary",),
        ),
    )(w2, w1, b1_2d, b2_2d)


def _head_kernel(h0_ref, m_ref, b2e_ref, out_ref):
    out_ref[...] = jax.lax.dot_general(
        h0_ref[...], m_ref[...], (((1,), (1,)), ((), ())),
        preferred_element_type=jnp.float32,
    ) + b2e_ref[...]


def _head(h0, m, b2e, n_tok, d_model, n_cols):
    return pl.pallas_call(
        _head_kernel,
        grid=(n_tok // _TM, n_cols // _TN),
        in_specs=[
            pl.BlockSpec((_TM, d_model), lambda t, v: (t, 0)),
            pl.BlockSpec((_TN, d_model), lambda t, v: (v, 0)),
            pl.BlockSpec((1, _TN), lambda t, v: (0, v)),
        ],
        out_specs=pl.BlockSpec((_TM, _TN), lambda t, v: (t, v)),
        out_shape=jax.ShapeDtypeStruct((n_tok, n_cols), jnp.float32),
        compiler_params=pltpu.CompilerParams(
            dimension_semantics=("arbitrary", "arbitrary"),
        ),
    )(h0, m, b2e)


def _head2(h0, m, b2e, n_tok, d_model, n_cols):
    return pl.pallas_call(
        _head_kernel,
        grid=(n_cols // _TN2, n_tok // _TM2),
        in_specs=[
            pl.BlockSpec((_TM2, d_model), lambda v, t: (t, 0)),
            pl.BlockSpec((_TN2, d_model), lambda v, t: (v, 0)),
            pl.BlockSpec((1, _TN2), lambda v, t: (0, v)),
        ],
        out_specs=pl.BlockSpec((_TM2, _TN2), lambda v, t: (t, v)),
        out_shape=jax.ShapeDtypeStruct((n_tok, n_cols), jnp.float32),
        compiler_params=pltpu.CompilerParams(
            dimension_semantics=("arbitrary", "arbitrary"),
        ),
    )(h0, m, b2e)


def _forward_1dev(x_flat, embed, W1, b1, W2, b2):
    (n_tok,) = x_flat.shape
    vocab, d_model = embed.shape
    h0 = _gather_rows(
        embed, x_flat.reshape(n_tok // 32, 32).astype(jnp.int32),
        n_tok, d_model)
    m, b2e = _precompute(W2, W1, b1.reshape(1, d_model), b2.reshape(1, vocab),
                         vocab, d_model)
    return _head(h0.astype(jnp.bfloat16), m, b2e, n_tok, d_model, vocab)


def _forward_2dev(x_all, embed_cols, W1, b1, W2_half, b2_half):
    (n_all,) = x_all.shape
    vocab, w = embed_cols.shape
    d_model = W1.shape[0]
    half_vocab, _ = W2_half.shape
    h0c = _gather_rows(
        embed_cols, x_all.reshape(n_all // 32, 32).astype(jnp.int32),
        n_all, w).astype(jnp.bfloat16)
    h0 = jax.lax.all_to_all(h0c, "d", split_axis=0, concat_axis=1, tiled=True)
    m_half, b2e_half = _precompute(W2_half, W1, b1.reshape(1, d_model),
                                   b2_half.reshape(1, half_vocab),
                                   half_vocab, d_model)
    m = jax.lax.all_gather(m_half, "d", axis=0, tiled=True)
    b2e = jax.lax.all_gather(b2e_half, "d", axis=1, tiled=True)
    return _head(h0, m, b2e, n_all // 2, d_model, vocab)


def kernel(x, embed, W1, b1, W2, b2):
    b, s = x.shape
    vocab, d_model = embed.shape
    n_tok = b * s
    devs = jax.devices()
    half = vocab // 2
    shardable = (len(devs) >= 2 and vocab % 2 == 0 and half % _TP == 0
                 and vocab % _TN == 0 and (n_tok // 2) % _TM == 0
                 and n_tok % 64 == 0 and d_model % 256 == 0)
    if not shardable:
        logits = _forward_1dev(x.reshape(n_tok), embed, W1, b1, W2, b2)
        return logits.reshape(b, s, vocab)
    mesh = jax.sharding.Mesh(np.array(devs[:2]), ("d",))
    p = jax.sharding.PartitionSpec
    logits = _shard_map(
        _forward_2dev,
        mesh=mesh,
        in_specs=(p(None), p(None, "d"), p(None, None), p(None),
                  p("d", None), p("d")),
        out_specs=p("d", None),
        check_vma=False,
    )(x.reshape(n_tok), embed, W1, b1, W2, b2)
    return logits.reshape(b, s, vocab)
